# initial kernel scaffold (unmeasured)
import jax
import jax.numpy as jnp
from jax import lax
from jax.experimental import pallas as pl
from jax.experimental.pallas import tpu as pltpu


def kernel(
    x,
):
    def body(*refs):
        pass

    out_shape = jax.ShapeDtypeStruct(..., jnp.float32)
    return pl.pallas_call(body, out_shape=out_shape)(...)


# baseline (device time: 10946 ns/iter reference)
import jax
import jax.numpy as jnp
from jax import lax
from jax.experimental import pallas as pl
from jax.experimental.pallas import tpu as pltpu

N_DEV = 4


def kernel(x):
    m_per, n = x.shape

    def body(x_ref, out_ref, comm_ref, send_sems, recv_sems):
        my_pos = lax.axis_index("i")
        left = (my_pos - 1) % N_DEV
        right = (my_pos + 1) % N_DEV

        barrier_sem = pltpu.get_barrier_semaphore()
        for nbr in [left, right]:
            pl.semaphore_signal(
                barrier_sem, inc=1,
                device_id=(nbr,), device_id_type=pl.DeviceIdType.MESH,
            )
        pl.semaphore_wait(barrier_sem, 2)

        vals = x_ref[:, :]
        maxv = jnp.max(vals, axis=0)
        row_ids = lax.broadcasted_iota(jnp.int32, (m_per, n), 0)
        masked = jnp.where(vals == maxv[None, :], row_ids, N_DEV * m_per)
        local_idx = jnp.min(masked, axis=0)
        gidx = (local_idx + my_pos * m_per).astype(jnp.float32)

        comm_ref[0, 0, :] = maxv
        comm_ref[0, 1, :] = gidx

        best_val = maxv
        best_idx = gidx

        for h in range(N_DEV - 1):
            rdma = pltpu.make_async_remote_copy(
                src_ref=comm_ref.at[h],
                dst_ref=comm_ref.at[h + 1],
                send_sem=send_sems.at[h],
                recv_sem=recv_sems.at[h + 1],
                device_id=(right,),
                device_id_type=pl.DeviceIdType.MESH,
            )
            rdma.start()
            rdma.wait()

            v = comm_ref[h + 1, 0, :]
            i = comm_ref[h + 1, 1, :]
            take = (v > best_val) | ((v == best_val) & (i < best_idx))
            best_val = jnp.where(take, v, best_val)
            best_idx = jnp.where(take, i, best_idx)

        out_ref[0, :] = best_val
        out_ref[1, :] = best_idx

    return pl.pallas_call(
        body,
        out_shape=jax.ShapeDtypeStruct((2, n), jnp.float32),
        in_specs=[pl.BlockSpec(memory_space=pltpu.VMEM)],
        out_specs=pl.BlockSpec(memory_space=pltpu.VMEM),
        scratch_shapes=[
            pltpu.VMEM((N_DEV, 2, n), jnp.float32),
            pltpu.SemaphoreType.DMA((N_DEV,)),
            pltpu.SemaphoreType.DMA((N_DEV,)),
        ],
        compiler_params=pltpu.CompilerParams(collective_id=0),
    )(x)


# device time: 7632 ns/iter; 1.4342x vs baseline; 1.4342x over previous
import jax
import jax.numpy as jnp
from jax import lax
from jax.experimental import pallas as pl
from jax.experimental.pallas import tpu as pltpu

N_DEV = 4


def kernel(x):
    m_per, n = x.shape

    def body(x_ref, out_ref, comm_ref, send_sems, recv_sems):
        my_pos = lax.axis_index("i")
        peers = [(my_pos + d) % N_DEV for d in (1, 2, 3)]

        barrier_sem = pltpu.get_barrier_semaphore()
        for p in peers:
            pl.semaphore_signal(
                barrier_sem, inc=1,
                device_id=(p,), device_id_type=pl.DeviceIdType.MESH,
            )
        pl.semaphore_wait(barrier_sem, N_DEV - 1)

        vals = x_ref[:, :]
        maxv = jnp.max(vals, axis=0)
        row_ids = lax.broadcasted_iota(jnp.int32, (m_per, n), 0)
        masked = jnp.where(vals == maxv[None, :], row_ids, N_DEV * m_per)
        local_idx = jnp.min(masked, axis=0)
        gidx = (local_idx + my_pos * m_per).astype(jnp.float32)

        comm_ref[0, 0, :] = maxv
        comm_ref[0, 1, :] = gidx

        rdmas = []
        for d, p in zip((1, 2, 3), peers):
            rdma = pltpu.make_async_remote_copy(
                src_ref=comm_ref.at[0],
                dst_ref=comm_ref.at[N_DEV - d],
                send_sem=send_sems.at[d],
                recv_sem=recv_sems.at[N_DEV - d],
                device_id=(p,),
                device_id_type=pl.DeviceIdType.MESH,
            )
            rdma.start()
            rdmas.append(rdma)

        best_val = maxv
        best_idx = gidx
        for d, rdma in zip((3, 1, 2), (rdmas[0], rdmas[2], rdmas[1])):
            rdma.wait_recv()
            v = comm_ref[d, 0, :]
            i = comm_ref[d, 1, :]
            take = (v > best_val) | ((v == best_val) & (i < best_idx))
            best_val = jnp.where(take, v, best_val)
            best_idx = jnp.where(take, i, best_idx)

        out_ref[0, :] = best_val
        out_ref[1, :] = best_idx

        for rdma in rdmas:
            rdma.wait_send()

    return pl.pallas_call(
        body,
        out_shape=jax.ShapeDtypeStruct((2, n), jnp.float32),
        in_specs=[pl.BlockSpec(memory_space=pltpu.VMEM)],
        out_specs=pl.BlockSpec(memory_space=pltpu.VMEM),
        scratch_shapes=[
            pltpu.VMEM((N_DEV, 2, n), jnp.float32),
            pltpu.SemaphoreType.DMA((N_DEV,)),
            pltpu.SemaphoreType.DMA((N_DEV,)),
        ],
        compiler_params=pltpu.CompilerParams(collective_id=0),
    )(x)


# device time: 7576 ns/iter; 1.4448x vs baseline; 1.0074x over previous
import jax
import jax.numpy as jnp
from jax import lax
from jax.experimental import pallas as pl
from jax.experimental.pallas import tpu as pltpu

N_DEV = 4
VAL, IDX = 0, 1


def kernel(x):
    m_per, n = x.shape

    def body(x_ref, out_ref, comm_ref, send_sems, recv_sems):
        my_pos = lax.axis_index("i")
        peers = [(my_pos + d) % N_DEV for d in (1, 2, 3)]

        barrier_sem = pltpu.get_barrier_semaphore()
        for p in peers:
            pl.semaphore_signal(
                barrier_sem, inc=1,
                device_id=(p,), device_id_type=pl.DeviceIdType.MESH,
            )

        vals = x_ref[:, :]
        maxv = jnp.max(vals, axis=0)
        comm_ref[VAL, 0, :] = maxv

        pl.semaphore_wait(barrier_sem, N_DEV - 1)

        def push(kind):
            out = []
            for d, p in ((2, peers[1]), (1, peers[0]), (3, peers[2])):
                rdma = pltpu.make_async_remote_copy(
                    src_ref=comm_ref.at[kind, 0],
                    dst_ref=comm_ref.at[kind, N_DEV - d],
                    send_sem=send_sems.at[kind, d],
                    recv_sem=recv_sems.at[kind, N_DEV - d],
                    device_id=(p,),
                    device_id_type=pl.DeviceIdType.MESH,
                )
                rdma.start()
                out.append(rdma)
            return out

        val_rdmas = push(VAL)

        row_ids = lax.broadcasted_iota(jnp.int32, (m_per, n), 0)
        masked = jnp.where(vals == maxv[None, :], row_ids, N_DEV * m_per)
        local_idx = jnp.min(masked, axis=0)
        gidx = (local_idx + my_pos * m_per).astype(jnp.float32)
        comm_ref[IDX, 0, :] = gidx

        idx_rdmas = push(IDX)

        best_val = maxv
        best_idx = gidx
        for slot, vr, ir in (
            (3, val_rdmas[1], idx_rdmas[1]),
            (1, val_rdmas[2], idx_rdmas[2]),
            (2, val_rdmas[0], idx_rdmas[0]),
        ):
            vr.wait_recv()
            ir.wait_recv()
            v = comm_ref[VAL, slot, :]
            i = comm_ref[IDX, slot, :]
            take = (v > best_val) | ((v == best_val) & (i < best_idx))
            best_val = jnp.where(take, v, best_val)
            best_idx = jnp.where(take, i, best_idx)

        out_ref[0, :] = best_val
        out_ref[1, :] = best_idx

        for rdma in val_rdmas + idx_rdmas:
            rdma.wait_send()

    return pl.pallas_call(
        body,
        out_shape=jax.ShapeDtypeStruct((2, n), jnp.float32),
        in_specs=[pl.BlockSpec(memory_space=pltpu.VMEM)],
        out_specs=pl.BlockSpec(memory_space=pltpu.VMEM),
        scratch_shapes=[
            pltpu.VMEM((2, N_DEV, n), jnp.float32),
            pltpu.SemaphoreType.DMA((2, N_DEV)),
            pltpu.SemaphoreType.DMA((2, N_DEV)),
        ],
        compiler_params=pltpu.CompilerParams(collective_id=0),
    )(x)


# device time: 7216 ns/iter; 1.5169x vs baseline; 1.0499x over previous
import jax
import jax.numpy as jnp
from jax import lax
from jax.experimental import pallas as pl
from jax.experimental.pallas import tpu as pltpu

N_DEV = 4
BLK = 32


def kernel(x):
    m_per, n = x.shape
    n_blk = m_per // BLK

    def body(x_ref, out_ref, comm_ref, send_sems, recv_sems):
        my_pos = lax.axis_index("i")
        peers = [(my_pos + d) % N_DEV for d in (1, 2, 3)]

        barrier_sem = pltpu.get_barrier_semaphore()
        for p in peers:
            pl.semaphore_signal(
                barrier_sem, inc=1,
                device_id=(p,), device_id_type=pl.DeviceIdType.MESH,
            )

        run_val = x_ref[0:BLK, :]
        run_blk = jnp.zeros((BLK, n), jnp.int32)
        for b in range(1, n_blk):
            v = x_ref[pl.ds(b * BLK, BLK), :]
            m = v > run_val
            run_val = jnp.where(m, v, run_val)
            run_blk = jnp.where(m, b, run_blk)

        maxv = jnp.max(run_val, axis=0)
        sub = lax.broadcasted_iota(jnp.int32, (BLK, n), 0)
        grow = run_blk * BLK + sub
        cand = jnp.where(run_val == maxv[None, :], grow, N_DEV * m_per)
        local_idx = jnp.min(cand, axis=0)
        gidx = (local_idx + my_pos * m_per).astype(jnp.float32)

        comm_ref[0, 0, :] = maxv
        comm_ref[0, 1, :] = gidx

        pl.semaphore_wait(barrier_sem, N_DEV - 1)

        rdmas = []
        for d, p in ((2, peers[1]), (1, peers[0]), (3, peers[2])):
            rdma = pltpu.make_async_remote_copy(
                src_ref=comm_ref.at[0],
                dst_ref=comm_ref.at[N_DEV - d],
                send_sem=send_sems.at[d],
                recv_sem=recv_sems.at[N_DEV - d],
                device_id=(p,),
                device_id_type=pl.DeviceIdType.MESH,
            )
            rdma.start()
            rdmas.append(rdma)

        best_val = maxv
        best_idx = gidx
        for slot, rdma in ((3, rdmas[1]), (1, rdmas[2]), (2, rdmas[0])):
            rdma.wait_recv()
            v = comm_ref[slot, 0, :]
            i = comm_ref[slot, 1, :]
            take = (v > best_val) | ((v == best_val) & (i < best_idx))
            best_val = jnp.where(take, v, best_val)
            best_idx = jnp.where(take, i, best_idx)

        out_ref[0, :] = best_val
        out_ref[1, :] = best_idx

        for rdma in rdmas:
            rdma.wait_send()

    return pl.pallas_call(
        body,
        out_shape=jax.ShapeDtypeStruct((2, n), jnp.float32),
        in_specs=[pl.BlockSpec(memory_space=pltpu.VMEM)],
        out_specs=pl.BlockSpec(memory_space=pltpu.VMEM),
        scratch_shapes=[
            pltpu.VMEM((N_DEV, 2, n), jnp.float32),
            pltpu.SemaphoreType.DMA((N_DEV,)),
            pltpu.SemaphoreType.DMA((N_DEV,)),
        ],
        compiler_params=pltpu.CompilerParams(collective_id=0),
    )(x)


# device time: 7162 ns/iter; 1.5283x vs baseline; 1.0075x over previous
import jax
import jax.numpy as jnp
from jax import lax
from jax.experimental import pallas as pl
from jax.experimental.pallas import tpu as pltpu

N_DEV = 4
BLK = 8


def kernel(x):
    m_per, n = x.shape
    n_blk = m_per // BLK

    def body(x_ref, out_ref, comm_ref, send_sems, recv_sems):
        my_pos = lax.axis_index("i")
        peers = [(my_pos + d) % N_DEV for d in (1, 2, 3)]

        barrier_sem = pltpu.get_barrier_semaphore()
        for p in peers:
            pl.semaphore_signal(
                barrier_sem, inc=1,
                device_id=(p,), device_id_type=pl.DeviceIdType.MESH,
            )

        run_val = x_ref[0:BLK, :]
        run_blk = jnp.zeros((BLK, n), jnp.int32)
        for b in range(1, n_blk):
            v = x_ref[pl.ds(b * BLK, BLK), :]
            m = v > run_val
            run_val = jnp.where(m, v, run_val)
            run_blk = jnp.where(m, b, run_blk)

        maxv = jnp.max(run_val, axis=0)
        sub = lax.broadcasted_iota(jnp.int32, (BLK, n), 0)
        grow = run_blk * BLK + sub
        cand = jnp.where(run_val == maxv[None, :], grow, N_DEV * m_per)
        local_idx = jnp.min(cand, axis=0)
        gidx = (local_idx + my_pos * m_per).astype(jnp.float32)

        comm_ref[0, 0, :] = maxv
        comm_ref[0, 1, :] = gidx

        pl.semaphore_wait(barrier_sem, N_DEV - 1)

        rdmas = []
        for d, p in ((2, peers[1]), (1, peers[0]), (3, peers[2])):
            rdma = pltpu.make_async_remote_copy(
                src_ref=comm_ref.at[0],
                dst_ref=comm_ref.at[N_DEV - d],
                send_sem=send_sems.at[d],
                recv_sem=recv_sems.at[N_DEV - d],
                device_id=(p,),
                device_id_type=pl.DeviceIdType.MESH,
            )
            rdma.start()
            rdmas.append(rdma)

        best_val = maxv
        best_idx = gidx
        for slot, rdma in ((3, rdmas[1]), (1, rdmas[2]), (2, rdmas[0])):
            rdma.wait_recv()
            v = comm_ref[slot, 0, :]
            i = comm_ref[slot, 1, :]
            take = (v > best_val) | ((v == best_val) & (i < best_idx))
            best_val = jnp.where(take, v, best_val)
            best_idx = jnp.where(take, i, best_idx)

        out_ref[0, :] = best_val
        out_ref[1, :] = best_idx

        for rdma in rdmas:
            rdma.wait_send()

    return pl.pallas_call(
        body,
        out_shape=jax.ShapeDtypeStruct((2, n), jnp.float32),
        in_specs=[pl.BlockSpec(memory_space=pltpu.VMEM)],
        out_specs=pl.BlockSpec(memory_space=pltpu.VMEM),
        scratch_shapes=[
            pltpu.VMEM((N_DEV, 2, n), jnp.float32),
            pltpu.SemaphoreType.DMA((N_DEV,)),
            pltpu.SemaphoreType.DMA((N_DEV,)),
        ],
        compiler_params=pltpu.CompilerParams(collective_id=0),
    )(x)
